# baseline (speedup 1.0000x reference)
import functools

import jax
import jax.numpy as jnp
from jax import lax
from jax.experimental import pallas as pl
from jax.experimental.pallas import tpu as pltpu
from jax.experimental.pallas import tpu_sc as plsc

_BATCH = 1024
_CLASSES = 1000
_DIM = 16
_OUT_W = _CLASSES * _DIM
_ROWS = 256

_NUM_CORES = 2
_NUM_SUBCORES = 16
_NW = _NUM_CORES * _NUM_SUBCORES
_BPW = _BATCH // _NW


def _fill_body(t0_ref, out_ref):
    out_ref[...] = jnp.broadcast_to(t0_ref[...], (_ROWS, _OUT_W))


def _sc_scatter_body(data_hbm, e1_hbm, out_hbm, data_v, e1_v, sem):
    wid = lax.axis_index("s") * _NUM_CORES + lax.axis_index("c")
    base = wid * _BPW
    pltpu.sync_copy(data_hbm.at[pl.ds(base, _BPW)], data_v)
    pltpu.sync_copy(e1_hbm, e1_v)
    copies = []
    for k in range(_BPW // 16):
        vec = data_v[pl.ds(k * 16, 16)]
        for t in range(16):
            off = vec[t] * _DIM
            row = base + k * 16 + t
            copies.append(
                pltpu.async_copy(e1_v, out_hbm.at[row, pl.ds(off, _DIM)], sem)
            )
    for cp in copies:
        cp.wait()


_sc_scatter = pl.kernel(
    _sc_scatter_body,
    out_type=(),
    mesh=plsc.VectorSubcoreMesh(core_axis_name="c", subcore_axis_name="s"),
    scratch_types=[
        pltpu.VMEM((_BPW,), jnp.int32),
        pltpu.VMEM((_DIM,), jnp.float32),
        pltpu.SemaphoreType.DMA,
    ],
)


def kernel(data, embed_weight):
    t0 = jnp.broadcast_to(embed_weight[0:1, :], (_CLASSES, _DIM)).reshape(1, _OUT_W)
    e1 = embed_weight[1]
    filled = pl.pallas_call(
        _fill_body,
        grid=(_BATCH // _ROWS,),
        in_specs=[pl.BlockSpec((1, _OUT_W), lambda i: (0, 0))],
        out_specs=pl.BlockSpec((_ROWS, _OUT_W), lambda i: (i, 0)),
        out_shape=jax.ShapeDtypeStruct((_BATCH, _OUT_W), jnp.float32),
    )(t0)
    out_ref = jax.new_ref(filled)
    _sc_scatter(data, e1, out_ref)
    return jax.freeze(out_ref)


# speedup vs baseline: 1.0268x; 1.0268x over previous
import functools

import jax
import jax.numpy as jnp
from jax import lax
from jax.experimental import pallas as pl
from jax.experimental.pallas import tpu as pltpu
from jax.experimental.pallas import tpu_sc as plsc

_BATCH = 1024
_CLASSES = 1000
_DIM = 16
_OUT_W = _CLASSES * _DIM
_ROWS = 64

_NUM_CORES = 2
_NUM_SUBCORES = 16
_NW = _NUM_CORES * _NUM_SUBCORES
_BPW = _BATCH // _NW


def _fill_body(t0_ref, out_ref):
    out_ref[...] = jnp.broadcast_to(t0_ref[...], (_ROWS, _OUT_W))


def _sc_scatter_body(data_hbm, e1_hbm, out_hbm, data_v, e1_v, sem):
    wid = lax.axis_index("s") * _NUM_CORES + lax.axis_index("c")
    base = wid * _BPW
    pltpu.sync_copy(data_hbm.at[pl.ds(base, _BPW)], data_v)
    pltpu.sync_copy(e1_hbm, e1_v)
    copies = []
    for k in range(_BPW // 16):
        vec = data_v[pl.ds(k * 16, 16)]
        for t in range(16):
            off = vec[t] * _DIM
            row = base + k * 16 + t
            copies.append(
                pltpu.async_copy(e1_v, out_hbm.at[row, pl.ds(off, _DIM)], sem)
            )
    for cp in copies:
        cp.wait()


_sc_scatter = pl.kernel(
    _sc_scatter_body,
    out_type=(),
    mesh=plsc.VectorSubcoreMesh(core_axis_name="c", subcore_axis_name="s"),
    scratch_types=[
        pltpu.VMEM((_BPW,), jnp.int32),
        pltpu.VMEM((_DIM,), jnp.float32),
        pltpu.SemaphoreType.DMA,
    ],
)


def kernel(data, embed_weight):
    t0 = jnp.broadcast_to(embed_weight[0:1, :], (_CLASSES, _DIM)).reshape(1, _OUT_W)
    e1 = embed_weight[1]
    filled = pl.pallas_call(
        _fill_body,
        grid=(_BATCH // _ROWS,),
        in_specs=[pl.BlockSpec((1, _OUT_W), lambda i: (0, 0))],
        out_specs=pl.BlockSpec((_ROWS, _OUT_W), lambda i: (i, 0)),
        out_shape=jax.ShapeDtypeStruct((_BATCH, _OUT_W), jnp.float32),
    )(t0)
    out_ref = jax.new_ref(filled)
    _sc_scatter(data, e1, out_ref)
    return jax.freeze(out_ref)
